# baseline (speedup 1.0000x reference)
import math

import jax
import jax.numpy as jnp
from jax import lax
from jax.experimental import pallas as pl
from jax.experimental.pallas import tpu as pltpu
from jax.experimental.pallas import tpu_sc as plsc

_LANES = 16


def _sc_embed_lookup(idx2d, embed, *, nw, nch, ch, d, scale):
    n = nw * nch * ch
    mesh = plsc.VectorSubcoreMesh(core_axis_name="c", subcore_axis_name="s")

    def body(idx_hbm, table_hbm, out_hbm, idx_v, rows_v, gsem):
        nc = 2
        wid = lax.axis_index("s") * nc + lax.axis_index("c")
        pltpu.sync_copy(idx_hbm.at[wid], idx_v)

        def chunk(j, carry):
            pltpu.async_copy(table_hbm.at[idx_v.at[j]], rows_v, gsem).wait()

            def srow(r, c2):
                for t in range(d // _LANES):
                    sl = pl.ds(t * _LANES, _LANES)
                    rows_v[r, sl] = rows_v[r, sl] * scale
                return c2

            lax.fori_loop(0, ch, srow, 0)
            pltpu.sync_copy(rows_v, out_hbm.at[pl.ds((wid * nch + j) * ch, ch)])
            return carry

        lax.fori_loop(0, nch, chunk, 0)

    f = pl.kernel(
        body,
        out_type=jax.ShapeDtypeStruct((n, d), jnp.float32),
        mesh=mesh,
        scratch_types=[
            pltpu.VMEM((nch, ch), jnp.int32),
            pltpu.VMEM((ch, d), jnp.float32),
            pltpu.SemaphoreType.DMA,
        ],
    )
    return f(idx2d, embed)


def kernel(x, embed):
    b, s = x.shape
    v, d = embed.shape
    n = b * s
    nw = 32
    ch = 128
    nch = n // (nw * ch)
    assert n == nw * nch * ch and d % _LANES == 0
    scale = math.sqrt(float(d))
    idx2d = x.reshape(nw, nch, ch).astype(jnp.int32)
    out = _sc_embed_lookup(idx2d, embed, nw=nw, nch=nch, ch=ch, d=d,
                           scale=scale)
    return out.reshape(b, s, d)


# speedup vs baseline: 1.2132x; 1.2132x over previous
import math

import jax
import jax.numpy as jnp
from jax import lax
from jax.experimental import pallas as pl
from jax.experimental.pallas import tpu as pltpu
from jax.experimental.pallas import tpu_sc as plsc

_LANES = 16
_NB = 5


def _sc_embed_lookup(idx3d, embed, *, nw, nch, ch, d, scale):
    n = nw * nch * ch
    mesh = plsc.VectorSubcoreMesh(core_axis_name="c", subcore_axis_name="s")
    ngrp = nch // _NB

    def body(idx_hbm, table_hbm, out_hbm, idx_v, rows_v, *sems):
        gsem, ssem = sems[:_NB], sems[_NB:]
        nc = 2
        wid = lax.axis_index("s") * nc + lax.axis_index("c")
        pltpu.sync_copy(idx_hbm.at[wid], idx_v)

        def start_gather(j, b):
            pltpu.make_async_copy(
                table_hbm.at[idx_v.at[j]], rows_v.at[b], gsem[b]).start()

        def wait_gather(j, b):
            pltpu.make_async_copy(
                table_hbm.at[idx_v.at[j]], rows_v.at[b], gsem[b]).wait()

        def start_store(j, b):
            pltpu.make_async_copy(
                rows_v.at[b], out_hbm.at[pl.ds((wid * nch + j) * ch, ch)],
                ssem[b]).start()

        def wait_store(j, b):
            pltpu.make_async_copy(
                rows_v.at[b], out_hbm.at[pl.ds((wid * nch + j) * ch, ch)],
                ssem[b]).wait()

        def scale_slot(b):
            @plsc.parallel_loop(0, ch, unroll=2)
            def _(r):
                for t in range(d // _LANES):
                    sl = pl.ds(t * _LANES, _LANES)
                    rows_v[b, r, sl] = rows_v[b, r, sl] * scale

        for b in range(_NB - 1):
            start_gather(b, b)

        for b in range(_NB):
            wait_gather(b, b)
            scale_slot(b)
            if b > 0:
                wait_store(b - 1, b - 1)
            start_gather(b + _NB - 1, (b + _NB - 1) % _NB)
            start_store(b, b)

        def grp(g, carry):
            for b in range(_NB):
                j = g * _NB + b
                wait_gather(j, b)
                scale_slot(b)
                wait_store(j - 1, (b - 1) % _NB)
                start_gather(j + _NB - 1, (b + _NB - 1) % _NB)
                start_store(j, b)
            return carry

        lax.fori_loop(1, ngrp - 1, grp, 0)

        j0 = (ngrp - 1) * _NB
        for b in range(_NB):
            j = j0 + b
            wait_gather(j, b)
            scale_slot(b)
            wait_store(j - 1, (b - 1) % _NB)
            if b == 0:
                start_gather(j + _NB - 1, (b + _NB - 1) % _NB)
            start_store(j, b)
        wait_store(nch - 1, (_NB - 1))

    f = pl.kernel(
        body,
        out_type=jax.ShapeDtypeStruct((n, d), jnp.float32),
        mesh=mesh,
        scratch_types=[
            pltpu.VMEM((nch, ch), jnp.int32),
            pltpu.VMEM((_NB, ch, d), jnp.float32),
        ] + [pltpu.SemaphoreType.DMA] * (2 * _NB),
    )
    return f(idx3d, embed)


def kernel(x, embed):
    b, s = x.shape
    v, d = embed.shape
    n = b * s
    nw = 32
    ch = 128
    nch = n // (nw * ch)
    assert n == nw * nch * ch and d % _LANES == 0
    assert nch % _NB == 0 and nch // _NB >= 2
    scale = math.sqrt(float(d))
    idx3d = x.reshape(nw, nch, ch).astype(jnp.int32)
    out = _sc_embed_lookup(idx3d, embed, nw=nw, nch=nch, ch=ch, d=d,
                           scale=scale)
    return out.reshape(b, s, d)


# speedup vs baseline: 2.1314x; 1.7568x over previous
import math

import jax
import jax.numpy as jnp
from jax import lax
from jax.experimental import pallas as pl
from jax.experimental.pallas import tpu as pltpu
from jax.experimental.pallas import tpu_sc as plsc

_LANES = 16
_NB = 8


def _sc_embed_lookup(x2d, embed, *, nw, nch, ch, d, scale):
    mesh = plsc.VectorSubcoreMesh(core_axis_name="c", subcore_axis_name="s")
    ngrp = nch // _NB

    def body(idx_hbm, table_hbm, out_hbm, idx_v, rows_v, *sems):
        gsem, ssem = sems[:_NB], sems[_NB:]
        nc = 2
        wid = lax.axis_index("s") * nc + lax.axis_index("c")
        pltpu.sync_copy(idx_hbm.at[pl.ds(wid * nch, nch)], idx_v)

        def start_gather(j, b):
            pltpu.make_async_copy(
                table_hbm.at[idx_v.at[j]], rows_v.at[b], gsem[b]).start()

        def wait_gather(j, b):
            pltpu.make_async_copy(
                table_hbm.at[idx_v.at[j]], rows_v.at[b], gsem[b]).wait()

        def start_store(j, b):
            pltpu.make_async_copy(
                rows_v.at[b], out_hbm.at[wid * nch + j], ssem[b]).start()

        def wait_store(j, b):
            pltpu.make_async_copy(
                rows_v.at[b], out_hbm.at[wid * nch + j], ssem[b]).wait()

        def scale_slot(b):
            @plsc.parallel_loop(0, ch, unroll=2)
            def _(r):
                for t in range(d // _LANES):
                    sl = pl.ds(t * _LANES, _LANES)
                    rows_v[b, r, sl] = rows_v[b, r, sl] * scale

        for b in range(_NB - 1):
            start_gather(b, b)

        for b in range(_NB):
            wait_gather(b, b)
            scale_slot(b)
            if b > 0:
                wait_store(b - 1, b - 1)
            start_gather(b + _NB - 1, (b + _NB - 1) % _NB)
            start_store(b, b)

        def grp(g, carry):
            for b in range(_NB):
                j = g * _NB + b
                wait_gather(j, b)
                scale_slot(b)
                wait_store(j - 1, (b - 1) % _NB)
                start_gather(j + _NB - 1, (b + _NB - 1) % _NB)
                start_store(j, b)
            return carry

        lax.fori_loop(1, ngrp - 1, grp, 0)

        j0 = (ngrp - 1) * _NB
        for b in range(_NB):
            j = j0 + b
            wait_gather(j, b)
            scale_slot(b)
            wait_store(j - 1, (b - 1) % _NB)
            if b == 0:
                start_gather(j + _NB - 1, (b + _NB - 1) % _NB)
            start_store(j, b)
        wait_store(nch - 1, _NB - 1)

    f = pl.kernel(
        body,
        out_type=jax.ShapeDtypeStruct((nw * nch, ch, d), jnp.float32),
        mesh=mesh,
        scratch_types=[
            pltpu.VMEM((nch, ch), jnp.int32),
            pltpu.VMEM((_NB, ch, d), jnp.float32),
        ] + [pltpu.SemaphoreType.DMA] * (2 * _NB),
    )
    return f(x2d, embed)


def kernel(x, embed):
    b, s = x.shape
    v, d = embed.shape
    nw = 32
    nch = b // nw
    ch = s
    assert b == nw * nch and d % _LANES == 0 and ch <= 128
    assert nch % _NB == 0 and nch // _NB >= 2
    scale = math.sqrt(float(d))
    return _sc_embed_lookup(x.astype(jnp.int32), embed, nw=nw, nch=nch,
                            ch=ch, d=d, scale=scale)


# speedup vs baseline: 3.7941x; 1.7801x over previous
import math

import jax
import jax.numpy as jnp
from jax import lax
from jax.experimental import pallas as pl
from jax.experimental.pallas import tpu as pltpu
from jax.experimental.pallas import tpu_sc as plsc

_LANES = 16
_NB = 5


def _sc_embed_lookup(xt, embed, *, nw, nch, ch, d, scale):
    mesh = plsc.VectorSubcoreMesh(core_axis_name="c", subcore_axis_name="s")
    ngrp = nch // _NB

    def body(idx_hbm, table_hbm, out_hbm, idx_v, rows_v, *sems):
        gsem, ssem = sems[:_NB], sems[_NB:]
        nc = 2
        wid = lax.axis_index("s") * nc + lax.axis_index("c")
        base = wid * ch
        pltpu.sync_copy(idx_hbm.at[:, pl.ds(base, ch)], idx_v)

        def start_gather(j, b):
            pltpu.make_async_copy(
                table_hbm.at[idx_v.at[j]], rows_v.at[b], gsem[b]).start()

        def wait_gather(j, b):
            pltpu.make_async_copy(
                table_hbm.at[idx_v.at[j]], rows_v.at[b], gsem[b]).wait()

        def start_store(j, b):
            pltpu.make_async_copy(
                rows_v.at[b], out_hbm.at[j].at[pl.ds(base, ch)],
                ssem[b]).start()

        def wait_store(j, b):
            pltpu.make_async_copy(
                rows_v.at[b], out_hbm.at[j].at[pl.ds(base, ch)],
                ssem[b]).wait()

        def scale_slot(b):
            @plsc.parallel_loop(0, ch, unroll=2)
            def _(r):
                for t in range(d // _LANES):
                    sl = pl.ds(t * _LANES, _LANES)
                    rows_v[b, r, sl] = rows_v[b, r, sl] * scale

        for b in range(_NB - 1):
            start_gather(b, b)

        for b in range(_NB):
            wait_gather(b, b)
            scale_slot(b)
            if b > 0:
                wait_store(b - 1, b - 1)
            start_gather(b + _NB - 1, (b + _NB - 1) % _NB)
            start_store(b, b)

        def grp(g, carry):
            for b in range(_NB):
                j = g * _NB + b
                wait_gather(j, b)
                scale_slot(b)
                wait_store(j - 1, (b - 1) % _NB)
                start_gather(j + _NB - 1, (b + _NB - 1) % _NB)
                start_store(j, b)
            return carry

        lax.fori_loop(1, ngrp - 1, grp, 0)

        j0 = (ngrp - 1) * _NB
        for b in range(_NB):
            j = j0 + b
            wait_gather(j, b)
            scale_slot(b)
            wait_store(j - 1, (b - 1) % _NB)
            if b == 0:
                start_gather(j + _NB - 1, (b + _NB - 1) % _NB)
            start_store(j, b)
        wait_store(nch - 1, _NB - 1)

    f = pl.kernel(
        body,
        out_type=jax.ShapeDtypeStruct((nch, nw * ch, d), jnp.float32),
        mesh=mesh,
        scratch_types=[
            pltpu.VMEM((nch, ch), jnp.int32),
            pltpu.VMEM((_NB, ch, d), jnp.float32),
        ] + [pltpu.SemaphoreType.DMA] * (2 * _NB),
    )
    return f(xt, embed)


def kernel(x, embed):
    b, s = x.shape
    v, d = embed.shape
    nw = 32
    ch = b // nw
    nch = s
    assert b == nw * ch and d % _LANES == 0 and ch <= 128
    assert nch % _NB == 0 and nch // _NB >= 2
    scale = math.sqrt(float(d))
    xt = jnp.swapaxes(x.astype(jnp.int32), 0, 1)
    out = _sc_embed_lookup(xt, embed, nw=nw, nch=nch, ch=ch, d=d, scale=scale)
    return jnp.swapaxes(out, 0, 1)
